# Initial kernel scaffold; baseline (speedup 1.0000x reference)
import jax
import jax.numpy as jnp
from jax.experimental import pallas as pl


def kernel(x, table):
    raise NotImplementedError("write your pallas kernel here")


# speedup vs baseline: 2.7168x; 2.7168x over previous
import functools

import jax
import jax.numpy as jnp
from jax import lax
from jax.experimental import pallas as pl
from jax.experimental.pallas import tpu as pltpu
from jax.experimental.pallas import tpu_sc as plsc

_DIM = 64
_SCALE = 8.0
_ROWS_BLOCK = 5000
_IDX_MINOR = 128


def _transform_body(table_ref, out_ref):
    pos = lax.broadcasted_iota(jnp.int32, (_ROWS_BLOCK, _DIM), 1).astype(jnp.float32) + 1.0
    out_ref[...] = table_ref[...] * _SCALE + pos


def _transform(table):
    vocab = table.shape[0]
    return pl.pallas_call(
        _transform_body,
        grid=(vocab // _ROWS_BLOCK,),
        in_specs=[pl.BlockSpec((_ROWS_BLOCK, _DIM), lambda i: (i, 0))],
        out_specs=pl.BlockSpec((_ROWS_BLOCK, _DIM), lambda i: (i, 0)),
        out_shape=jax.ShapeDtypeStruct((vocab, _DIM), jnp.float32),
    )(table)


@functools.lru_cache(maxsize=None)
def _make_gather(n_rows, vocab):
    info = plsc.get_sparse_core_info()
    nc, ns = info.num_cores, info.num_subcores
    nw = nc * ns
    chunks_per_w = n_rows // (nw * _IDX_MINOR)
    mesh = plsc.VectorSubcoreMesh(core_axis_name="c", subcore_axis_name="s")

    @functools.partial(
        pl.kernel,
        mesh=mesh,
        compiler_params=pltpu.CompilerParams(use_tc_tiling_on_sc=False),
        out_type=jax.ShapeDtypeStruct((n_rows, _DIM), jnp.float32),
        scratch_types=[
            pltpu.VMEM((chunks_per_w, _IDX_MINOR), jnp.int32),
            pltpu.VMEM((_IDX_MINOR, _DIM), jnp.float32),
            pltpu.SemaphoreType.DMA,
        ],
    )
    def k(idx_hbm, table_hbm, out_hbm, idx_v, rows_v, sem):
        wid = lax.axis_index("s") * nc + lax.axis_index("c")
        pltpu.sync_copy(idx_hbm.at[wid], idx_v)

        def body(j, carry):
            pltpu.async_copy(table_hbm.at[idx_v.at[j]], rows_v, sem).wait()
            pltpu.sync_copy(
                rows_v,
                out_hbm.at[pl.ds((wid * chunks_per_w + j) * _IDX_MINOR, _IDX_MINOR)],
            )
            return carry

        lax.fori_loop(0, chunks_per_w, body, 0)

    return k


def kernel(x, table):
    b, l = x.shape
    nw = plsc.get_sparse_core_info().num_cores * plsc.get_sparse_core_info().num_subcores
    idx = x.reshape(nw, -1, _IDX_MINOR).astype(jnp.int32)
    table2 = _transform(table)
    out = _make_gather(b * l, table.shape[0])(idx, table2)
    return out.reshape(1, b, l, _DIM)
